# baseline (speedup 1.0000x reference)
import functools

import jax
import jax.numpy as jnp
from jax import lax
from jax.experimental import pallas as pl
from jax.experimental.pallas import tpu as pltpu
from jax.experimental.pallas import tpu_sc as plsc

D = 64


@functools.lru_cache(maxsize=None)
def _make_gather(B: int):
    info = plsc.get_sparse_core_info()
    NC, NS = info.num_cores, info.num_subcores
    NW = NC * NS
    assert B % (8 * NW) == 0
    b_per_w = B // NW
    mesh = plsc.VectorSubcoreMesh(core_axis_name="c", subcore_axis_name="s")

    @functools.partial(
        pl.kernel,
        mesh=mesh,
        compiler_params=pltpu.CompilerParams(use_tc_tiling_on_sc=False),
        out_type=(
            jax.ShapeDtypeStruct((B, D), jnp.float32),
            jax.ShapeDtypeStruct((B, D), jnp.float32),
        ),
        scratch_types=[
            pltpu.VMEM((b_per_w,), jnp.int32),
            pltpu.VMEM((b_per_w,), jnp.int32),
            pltpu.VMEM((b_per_w, D), jnp.float32),
            pltpu.VMEM((b_per_w, D), jnp.float32),
            pltpu.SemaphoreType.DMA,
            pltpu.SemaphoreType.DMA,
        ],
    )
    def gather_k(ut_hbm, mt_hbm, uid_hbm, mid_hbm, u_out, m_out,
                 uidx_v, midx_v, urows_v, mrows_v, usem, msem):
        wid = lax.axis_index("s") * NC + lax.axis_index("c")
        base = wid * b_per_w
        pltpu.sync_copy(uid_hbm.at[pl.ds(base, b_per_w)], uidx_v)
        pltpu.sync_copy(mid_hbm.at[pl.ds(base, b_per_w)], midx_v)
        ucp = pltpu.async_copy(ut_hbm.at[uidx_v], urows_v, usem)
        mcp = pltpu.async_copy(mt_hbm.at[midx_v], mrows_v, msem)
        ucp.wait()
        pltpu.sync_copy(urows_v, u_out.at[pl.ds(base, b_per_w)])
        mcp.wait()
        pltpu.sync_copy(mrows_v, m_out.at[pl.ds(base, b_per_w)])

    return gather_k


def _tower(e, W1, b1, W2, b2):
    h = jnp.maximum(
        jnp.dot(e, W1, preferred_element_type=jnp.float32) + b1, 0.0)
    o = jnp.dot(h, W2, preferred_element_type=jnp.float32) + b2
    sq = jnp.sum(o * o, axis=1, keepdims=True)
    return o * lax.rsqrt(jnp.maximum(sq, 1e-12))


def _mlp_body(eu_ref, em_ref, uW1_ref, ub1_ref, uW2_ref, ub2_ref,
              mW1_ref, mb1_ref, mW2_ref, mb2_ref, out_ref):
    out_ref[0] = _tower(eu_ref[...], uW1_ref[...], ub1_ref[...],
                        uW2_ref[...], ub2_ref[...])
    out_ref[1] = _tower(em_ref[...], mW1_ref[...], mb1_ref[...],
                        mW2_ref[...], mb2_ref[...])


@functools.lru_cache(maxsize=None)
def _make_mlp(B: int, bs: int):
    grid = B // bs
    w_spec = pl.BlockSpec((D, D), lambda i: (0, 0))
    b_spec = pl.BlockSpec((1, D), lambda i: (0, 0))
    e_spec = pl.BlockSpec((bs, D), lambda i: (i, 0))
    return pl.pallas_call(
        _mlp_body,
        grid=(grid,),
        in_specs=[e_spec, e_spec,
                  w_spec, b_spec, w_spec, b_spec,
                  w_spec, b_spec, w_spec, b_spec],
        out_specs=pl.BlockSpec((2, bs, D), lambda i: (0, i, 0)),
        out_shape=jax.ShapeDtypeStruct((2, B, D), jnp.float32),
    )


@jax.jit
def kernel(user_ids, movie_ids, user_table, movie_table,
           uW1, ub1, uW2, ub2, mW1, mb1, mW2, mb2):
    B = user_ids.shape[0]
    eu, em = _make_gather(B)(
        user_table, movie_table,
        user_ids.astype(jnp.int32), movie_ids.astype(jnp.int32))
    return _make_mlp(B, 2048)(
        eu, em,
        uW1, ub1.reshape(1, D), uW2, ub2.reshape(1, D),
        mW1, mb1.reshape(1, D), mW2, mb2.reshape(1, D))
